# baseline (speedup 1.0000x reference)
import functools

import jax
import jax.numpy as jnp
import numpy as np
from jax import lax
from jax.experimental import pallas as pl
from jax.experimental.pallas import tpu as pltpu
from jax.experimental.pallas import tpu_sc as plsc

BS, LQ, D = 4, 5440, 256
NH, NL, NPT = 8, 4, 4
HD = D // NH
SSH = np.array([[64, 64], [32, 32], [16, 16], [8, 8]], dtype=np.int64)
STARTS = np.concatenate([[0], np.cumsum(SSH[:, 0] * SSH[:, 1])])
K128 = NH * NL * NPT
KT = 4 * K128
BLK = 544
NBLK = LQ // BLK

_l_of_k = (np.arange(K128) // NPT) % NL
_W_of_k = SSH[_l_of_k, 1].astype(np.float32)
_H_of_k = SSH[_l_of_k, 0].astype(np.float32)
_START_of_k = STARTS[_l_of_k].astype(np.int32)
_H_head_of_k = (np.arange(K128) // (NL * NPT)).astype(np.int32)
_PX = np.zeros((NL, K128), np.float32)
_PX[_l_of_k, np.arange(K128)] = _W_of_k
_PY = np.zeros((NL, K128), np.float32)
_PY[_l_of_k, np.arange(K128)] = _H_of_k
_G = np.kron(np.eye(NH, dtype=np.float32), np.ones((NL * NPT, NL * NPT), np.float32))


def _prep_body(q_ref, rx_ref, ry_ref, wval_ref, bval_ref, wox_ref, box_ref,
               woy_ref, boy_ref, wattn_ref, battn_ref, g_ref, px_ref, py_ref,
               tabf_ref, tabi_ref,
               val_ref, idx_ref, wt_ref):
    b = pl.program_id(0)
    q = q_ref[0]
    val_ref[0] = (jnp.dot(q, wval_ref[...], preferred_element_type=jnp.float32, precision=lax.Precision.HIGHEST) + bval_ref[...]).astype(jnp.bfloat16)
    offx = jnp.dot(q, wox_ref[...], preferred_element_type=jnp.float32, precision=lax.Precision.HIGHEST) + box_ref[...]
    offy = jnp.dot(q, woy_ref[...], preferred_element_type=jnp.float32, precision=lax.Precision.HIGHEST) + boy_ref[...]
    a = jnp.dot(q, wattn_ref[...], preferred_element_type=jnp.float32, precision=lax.Precision.HIGHEST) + battn_ref[...]
    a = a - jnp.max(a, axis=-1, keepdims=True)
    e = jnp.exp(a)
    aw = e / jnp.dot(e, g_ref[...], preferred_element_type=jnp.float32, precision=lax.Precision.HIGHEST)

    px = jnp.dot(rx_ref[0], px_ref[...], preferred_element_type=jnp.float32, precision=lax.Precision.HIGHEST) + offx - 0.5
    py = jnp.dot(ry_ref[0], py_ref[...], preferred_element_type=jnp.float32, precision=lax.Precision.HIGHEST) + offy - 0.5
    x0 = jnp.floor(px)
    y0 = jnp.floor(py)
    fx = px - x0
    fy = py - y0

    wc = tabf_ref[0:1, :]
    hc = tabf_ref[1:2, :]
    wci = tabi_ref[0:1, :]
    sc = tabi_ref[1:2, :]
    hk = tabi_ref[2:3, :]
    for c in range(4):
        dx, dy = float(c & 1), float(c >> 1)
        ix = x0 + dx
        iy = y0 + dy
        valid = ((ix >= 0.0) & (ix <= wc - 1.0) & (iy >= 0.0) & (iy <= hc - 1.0))
        ixc = jnp.clip(ix, 0.0, wc - 1.0).astype(jnp.int32)
        iyc = jnp.clip(iy, 0.0, hc - 1.0).astype(jnp.int32)
        pos = sc + iyc * wci + ixc
        row = (b * LQ + pos) * NH + hk
        wx = fx if dx else 1.0 - fx
        wy = fy if dy else 1.0 - fy
        idx_ref[0, :, c * K128:(c + 1) * K128] = row
        wt_ref[0, :, c * K128:(c + 1) * K128] = aw * wx * wy * valid.astype(jnp.float32)


def _prep(query, refx, refy, W_val, b_val, Wox, box, Woy, boy, W_attn, b_attn):
    full = lambda shp: pl.BlockSpec(shp, lambda b, i: tuple(0 for _ in shp))
    return pl.pallas_call(
        _prep_body,
        grid=(BS, NBLK),
        in_specs=[
            pl.BlockSpec((1, BLK, D), lambda b, i: (b, i, 0)),
            pl.BlockSpec((1, BLK, NL), lambda b, i: (b, i, 0)),
            pl.BlockSpec((1, BLK, NL), lambda b, i: (b, i, 0)),
            full((D, D)), full((1, D)),
            full((D, K128)), full((1, K128)),
            full((D, K128)), full((1, K128)),
            full((D, K128)), full((1, K128)),
            full((K128, K128)), full((NL, K128)), full((NL, K128)),
            full((2, K128)), full((3, K128)),
        ],
        out_specs=[
            pl.BlockSpec((1, BLK, D), lambda b, i: (b, i, 0)),
            pl.BlockSpec((1, BLK, KT), lambda b, i: (b, i, 0)),
            pl.BlockSpec((1, BLK, KT), lambda b, i: (b, i, 0)),
        ],
        out_shape=[
            jax.ShapeDtypeStruct((BS, LQ, D), jnp.bfloat16),
            jax.ShapeDtypeStruct((BS, LQ, KT), jnp.int32),
            jax.ShapeDtypeStruct((BS, LQ, KT), jnp.float32),
        ],
    )(query, refx, refy, W_val, b_val, Wox, box, Woy, boy, W_attn, b_attn,
      jnp.asarray(_G), jnp.asarray(_PX), jnp.asarray(_PY),
      jnp.asarray(np.stack([_W_of_k, _H_of_k])),
      jnp.asarray(np.stack([_W_of_k.astype(np.int32), _START_of_k, _H_head_of_k])))


NW = 32
QPW = (BS * LQ) // NW
CH = 4
NIT = QPW // CH


def _sample_body(value_hbm, idx_hbm, wt_hbm, out_hbm, idx_v, wt_v, rows_v, out_v,
                 gsem0, gsem1, psem):
    wid = lax.axis_index("s") * 2 + lax.axis_index("c")
    qbase0 = wid * QPW
    gsems = (gsem0, gsem1)

    def idxwt_descs(it, slot):
        qb = qbase0 + it * CH
        return (pltpu.make_async_copy(idx_hbm.at[pl.ds(qb * 4, CH * 4)],
                                      idx_v.at[slot], psem),
                pltpu.make_async_copy(wt_hbm.at[pl.ds(qb, CH)],
                                      wt_v.at[slot], psem))

    def gather_descs(slot):
        return [pltpu.make_async_copy(value_hbm.at[idx_v.at[slot, j]],
                                      rows_v.at[slot, pl.ds(j * 128, 128)],
                                      gsems[slot])
                for j in range(CH * 4)]

    def compute(it, slot):
        for qq in range(CH):
            def h_body(h, c2):
                acc0 = jnp.zeros((16,), jnp.float32)
                acc1 = jnp.zeros((16,), jnp.float32)
                for c in range(4):
                    wv = wt_v[slot, qq, pl.ds(c * K128 + h * 16, 16)]
                    for t in range(16):
                        r = qq * KT + c * K128 + h * 16 + t
                        v = rows_v[slot, r, :]
                        lo = lax.bitcast_convert_type(v << 16, jnp.float32)
                        hi = lax.bitcast_convert_type(v & jnp.int32(-65536),
                                                      jnp.float32)
                        acc0 = acc0 + wv[t] * lo
                        acc1 = acc1 + wv[t] * hi
                out_v[qq, pl.ds(h * HD, 16)] = acc0
                out_v[qq, pl.ds(h * HD + 16, 16)] = acc1
                return c2
            lax.fori_loop(0, NH, h_body, 0)
        pltpu.sync_copy(out_v, out_hbm.at[pl.ds(qbase0 + it * CH, CH)])

    pltpu.sync_copy(idx_hbm.at[pl.ds(qbase0 * 4, CH * 4)], idx_v.at[0])
    pltpu.sync_copy(wt_hbm.at[pl.ds(qbase0, CH)], wt_v.at[0])
    for d in gather_descs(0):
        d.start()
    for d in idxwt_descs(1, 1):
        d.start()

    def body2(it2, carry):
        for slot in (0, 1):
            it = it2 * 2 + slot
            nxt = 1 - slot

            @pl.when(it + 1 < NIT)
            def _():
                for d in idxwt_descs(it + 1, nxt):
                    d.wait()
                for d in gather_descs(nxt):
                    d.start()

            for d in gather_descs(slot):
                d.wait()

            compute(it, slot)

            @pl.when(it + 2 < NIT)
            def _():
                for d in idxwt_descs(it + 2, slot):
                    d.start()
        return carry

    lax.fori_loop(0, NIT // 2, body2, 0)


@functools.cache
def _make_sample():
    return pl.kernel(
        _sample_body,
        mesh=plsc.VectorSubcoreMesh(core_axis_name="c", subcore_axis_name="s"),
        compiler_params=pltpu.CompilerParams(use_tc_tiling_on_sc=False),
        out_type=jax.ShapeDtypeStruct((BS * LQ, D), jnp.float32),
        scratch_types=[
            pltpu.VMEM((2, CH * 4, 128), jnp.int32),
            pltpu.VMEM((2, CH, KT), jnp.float32),
            pltpu.VMEM((2, CH * KT, 16), jnp.int32),
            pltpu.VMEM((CH, D), jnp.float32),
            pltpu.SemaphoreType.DMA,
            pltpu.SemaphoreType.DMA,
            pltpu.SemaphoreType.DMA,
        ],
    )


def _outproj_body(a_ref, q_ref, w_ref, b_ref, o_ref):
    o_ref[...] = (q_ref[...] + b_ref[...]
                  + jnp.dot(a_ref[...], w_ref[...], preferred_element_type=jnp.float32, precision=lax.Precision.HIGHEST))


def _outproj(attn, query2, W_out, b_out):
    return pl.pallas_call(
        _outproj_body,
        grid=(BS * NBLK,),
        in_specs=[
            pl.BlockSpec((BLK, D), lambda i: (i, 0)),
            pl.BlockSpec((BLK, D), lambda i: (i, 0)),
            pl.BlockSpec((D, D), lambda i: (0, 0)),
            pl.BlockSpec((1, D), lambda i: (0, 0)),
        ],
        out_specs=pl.BlockSpec((BLK, D), lambda i: (i, 0)),
        out_shape=jax.ShapeDtypeStruct((BS * LQ, D), jnp.float32),
    )(attn, query2, W_out, b_out)


_EVOD = np.concatenate([h * HD + np.concatenate([np.arange(0, HD, 2), np.arange(1, HD, 2)])
                        for h in range(NH)])


def kernel(query, reference_points, spatial_shapes, W_off, b_off, W_attn, b_attn,
           W_val, b_val, W_out, b_out):
    del spatial_shapes
    refx = reference_points[..., 0]
    refy = reference_points[..., 1]
    Wox, Woy = W_off[:, 0::2], W_off[:, 1::2]
    box, boy = b_off[0::2][None, :], b_off[1::2][None, :]
    value, idx, wt = _prep(query, refx, refy, W_val, b_val[None, :], Wox, box,
                           Woy, boy, W_attn, b_attn[None, :])
    vpacked = lax.bitcast_convert_type(value.reshape(BS * LQ * NH, 16, 2),
                                       jnp.int32)
    attn = _make_sample()(vpacked,
                          idx.reshape(BS * LQ * 4, K128),
                          wt.reshape(BS * LQ, KT))
    out = _outproj(attn, query.reshape(BS * LQ, D), W_out[_EVOD], b_out[None, :])
    return out.reshape(BS, LQ, D)


# speedup vs baseline: 5.3661x; 5.3661x over previous
import functools

import jax
import jax.numpy as jnp
import numpy as np
from jax import lax
from jax.experimental import pallas as pl
from jax.experimental.pallas import tpu as pltpu
from jax.experimental.pallas import tpu_sc as plsc

BS, LQ, D = 4, 5440, 256
NH, NL, NPT = 8, 4, 4
HD = D // NH
SSH = np.array([[64, 64], [32, 32], [16, 16], [8, 8]], dtype=np.int64)
STARTS = np.concatenate([[0], np.cumsum(SSH[:, 0] * SSH[:, 1])])
K128 = NH * NL * NPT
KT = 4 * K128
BLK = 544
NBLK = LQ // BLK

_l_of_k = (np.arange(K128) // NPT) % NL
_W_of_k = SSH[_l_of_k, 1].astype(np.float32)
_H_of_k = SSH[_l_of_k, 0].astype(np.float32)
_START_of_k = STARTS[_l_of_k].astype(np.int32)
_H_head_of_k = (np.arange(K128) // (NL * NPT)).astype(np.int32)
_PX = np.zeros((NL, K128), np.float32)
_PX[_l_of_k, np.arange(K128)] = _W_of_k
_PY = np.zeros((NL, K128), np.float32)
_PY[_l_of_k, np.arange(K128)] = _H_of_k
_G = np.kron(np.eye(NH, dtype=np.float32), np.ones((NL * NPT, NL * NPT), np.float32))


def _prep_body(q_ref, rx_ref, ry_ref, wval_ref, bval_ref, wox_ref, box_ref,
               woy_ref, boy_ref, wattn_ref, battn_ref, g_ref, px_ref, py_ref,
               tabf_ref, tabi_ref,
               val_ref, idx_ref, wt_ref):
    b = pl.program_id(0)
    q = q_ref[0]
    val = jnp.dot(q, wval_ref[...], preferred_element_type=jnp.float32, precision=lax.Precision.HIGHEST) + bval_ref[...]
    u = lax.bitcast_convert_type(val, jnp.int32)
    rnd = lambda x: lax.shift_right_logical(
        x + 0x7FFF + (lax.shift_right_logical(x, 16) & 1), 16)
    val_ref[0] = rnd(u[:, :K128]) | (rnd(u[:, K128:]) << 16)
    offx = jnp.dot(q, wox_ref[...], preferred_element_type=jnp.float32, precision=lax.Precision.HIGHEST) + box_ref[...]
    offy = jnp.dot(q, woy_ref[...], preferred_element_type=jnp.float32, precision=lax.Precision.HIGHEST) + boy_ref[...]
    a = jnp.dot(q, wattn_ref[...], preferred_element_type=jnp.float32, precision=lax.Precision.HIGHEST) + battn_ref[...]
    a = a - jnp.max(a, axis=-1, keepdims=True)
    e = jnp.exp(a)
    aw = e / jnp.dot(e, g_ref[...], preferred_element_type=jnp.float32, precision=lax.Precision.HIGHEST)

    px = jnp.dot(rx_ref[0], px_ref[...], preferred_element_type=jnp.float32, precision=lax.Precision.HIGHEST) + offx - 0.5
    py = jnp.dot(ry_ref[0], py_ref[...], preferred_element_type=jnp.float32, precision=lax.Precision.HIGHEST) + offy - 0.5
    x0 = jnp.floor(px)
    y0 = jnp.floor(py)
    fx = px - x0
    fy = py - y0

    wc = tabf_ref[0:1, :]
    hc = tabf_ref[1:2, :]
    wci = tabi_ref[0:1, :]
    sc = tabi_ref[1:2, :]
    hk = tabi_ref[2:3, :]
    for c in range(4):
        dx, dy = float(c & 1), float(c >> 1)
        ix = x0 + dx
        iy = y0 + dy
        valid = ((ix >= 0.0) & (ix <= wc - 1.0) & (iy >= 0.0) & (iy <= hc - 1.0))
        ixc = jnp.clip(ix, 0.0, wc - 1.0).astype(jnp.int32)
        iyc = jnp.clip(iy, 0.0, hc - 1.0).astype(jnp.int32)
        pos = sc + iyc * wci + ixc
        row = (b * LQ + pos) * NH + hk
        wx = fx if dx else 1.0 - fx
        wy = fy if dy else 1.0 - fy
        idx_ref[0, :, c * K128:(c + 1) * K128] = row
        wt_ref[0, :, c * K128:(c + 1) * K128] = aw * wx * wy * valid.astype(jnp.float32)


def _prep(query, refx, refy, W_val, b_val, Wox, box, Woy, boy, W_attn, b_attn):
    full = lambda shp: pl.BlockSpec(shp, lambda b, i: tuple(0 for _ in shp))
    return pl.pallas_call(
        _prep_body,
        grid=(BS, NBLK),
        in_specs=[
            pl.BlockSpec((1, BLK, D), lambda b, i: (b, i, 0)),
            pl.BlockSpec((1, BLK, NL), lambda b, i: (b, i, 0)),
            pl.BlockSpec((1, BLK, NL), lambda b, i: (b, i, 0)),
            full((D, D)), full((1, D)),
            full((D, K128)), full((1, K128)),
            full((D, K128)), full((1, K128)),
            full((D, K128)), full((1, K128)),
            full((K128, K128)), full((NL, K128)), full((NL, K128)),
            full((2, K128)), full((3, K128)),
        ],
        out_specs=[
            pl.BlockSpec((1, BLK, K128), lambda b, i: (b, i, 0)),
            pl.BlockSpec((1, BLK, KT), lambda b, i: (b, i, 0)),
            pl.BlockSpec((1, BLK, KT), lambda b, i: (b, i, 0)),
        ],
        out_shape=[
            jax.ShapeDtypeStruct((BS, LQ, K128), jnp.int32),
            jax.ShapeDtypeStruct((BS, LQ, KT), jnp.int32),
            jax.ShapeDtypeStruct((BS, LQ, KT), jnp.float32),
        ],
    )(query, refx, refy, W_val, b_val, Wox, box, Woy, boy, W_attn, b_attn,
      jnp.asarray(_G), jnp.asarray(_PX), jnp.asarray(_PY),
      jnp.asarray(np.stack([_W_of_k, _H_of_k])),
      jnp.asarray(np.stack([_W_of_k.astype(np.int32), _START_of_k, _H_head_of_k])))


NW = 32
QPW = (BS * LQ) // NW
CH = 4
NIT = QPW // CH


def _sample_body(value_hbm, idx_hbm, wt_hbm, out_hbm, idx_v, wt_v, rows_v, out_v,
                 gsem0, gsem1, psem):
    wid = lax.axis_index("s") * 2 + lax.axis_index("c")
    qbase0 = wid * QPW
    gsems = (gsem0, gsem1)

    def idxwt_descs(it, slot):
        qb = qbase0 + it * CH
        return (pltpu.make_async_copy(idx_hbm.at[pl.ds(qb * 4, CH * 4)],
                                      idx_v.at[slot], psem),
                pltpu.make_async_copy(wt_hbm.at[pl.ds(qb, CH)],
                                      wt_v.at[slot], psem))

    def gather_descs(slot):
        return [pltpu.make_async_copy(value_hbm.at[idx_v.at[slot, j]],
                                      rows_v.at[slot, pl.ds(j * 128, 128)],
                                      gsems[slot])
                for j in range(CH * 4)]

    def compute(it, slot):
        for qq in range(CH):
            def h_body(h, c2):
                acc0 = jnp.zeros((16,), jnp.float32)
                acc1 = jnp.zeros((16,), jnp.float32)
                for c in range(4):
                    wv = wt_v[slot, qq, pl.ds(c * K128 + h * 16, 16)]
                    for t in range(16):
                        r = qq * KT + c * K128 + h * 16 + t
                        v = rows_v[slot, r, :]
                        lo = lax.bitcast_convert_type(v << 16, jnp.float32)
                        hi = lax.bitcast_convert_type(v & jnp.int32(-65536),
                                                      jnp.float32)
                        acc0 = acc0 + wv[t] * lo
                        acc1 = acc1 + wv[t] * hi
                out_v[qq, pl.ds(h * HD, 16)] = acc0
                out_v[qq, pl.ds(h * HD + 16, 16)] = acc1
                return c2
            lax.fori_loop(0, NH, h_body, 0)
        pltpu.sync_copy(out_v, out_hbm.at[pl.ds(qbase0 + it * CH, CH)])

    pltpu.sync_copy(idx_hbm.at[pl.ds(qbase0 * 4, CH * 4)], idx_v.at[0])
    pltpu.sync_copy(wt_hbm.at[pl.ds(qbase0, CH)], wt_v.at[0])
    for d in gather_descs(0):
        d.start()
    for d in idxwt_descs(1, 1):
        d.start()

    def body2(it2, carry):
        for slot in (0, 1):
            it = it2 * 2 + slot
            nxt = 1 - slot

            @pl.when(it + 1 < NIT)
            def _():
                for d in idxwt_descs(it + 1, nxt):
                    d.wait()
                for d in gather_descs(nxt):
                    d.start()

            for d in gather_descs(slot):
                d.wait()

            compute(it, slot)

            @pl.when(it + 2 < NIT)
            def _():
                for d in idxwt_descs(it + 2, slot):
                    d.start()
        return carry

    lax.fori_loop(0, NIT // 2, body2, 0)


@functools.cache
def _make_sample():
    return pl.kernel(
        _sample_body,
        mesh=plsc.VectorSubcoreMesh(core_axis_name="c", subcore_axis_name="s"),
        compiler_params=pltpu.CompilerParams(use_tc_tiling_on_sc=False),
        out_type=jax.ShapeDtypeStruct((BS * LQ, D), jnp.float32),
        scratch_types=[
            pltpu.VMEM((2, CH * 4, 128), jnp.int32),
            pltpu.VMEM((2, CH, KT), jnp.float32),
            pltpu.VMEM((2, CH * KT, 16), jnp.int32),
            pltpu.VMEM((CH, D), jnp.float32),
            pltpu.SemaphoreType.DMA,
            pltpu.SemaphoreType.DMA,
            pltpu.SemaphoreType.DMA,
        ],
    )


def _outproj_body(a_ref, q_ref, w_ref, b_ref, o_ref):
    o_ref[...] = (q_ref[...] + b_ref[...]
                  + jnp.dot(a_ref[...], w_ref[...], preferred_element_type=jnp.float32, precision=lax.Precision.HIGHEST))


def _outproj(attn, query2, W_out, b_out):
    return pl.pallas_call(
        _outproj_body,
        grid=(BS * NBLK,),
        in_specs=[
            pl.BlockSpec((BLK, D), lambda i: (i, 0)),
            pl.BlockSpec((BLK, D), lambda i: (i, 0)),
            pl.BlockSpec((D, D), lambda i: (0, 0)),
            pl.BlockSpec((1, D), lambda i: (0, 0)),
        ],
        out_specs=pl.BlockSpec((BLK, D), lambda i: (i, 0)),
        out_shape=jax.ShapeDtypeStruct((BS * LQ, D), jnp.float32),
    )(attn, query2, W_out, b_out)


_LOHI = np.concatenate([np.arange(NH * HD).reshape(NH, 2, 16)[:, 0].ravel(),
                        np.arange(NH * HD).reshape(NH, 2, 16)[:, 1].ravel()])


def kernel(query, reference_points, spatial_shapes, W_off, b_off, W_attn, b_attn,
           W_val, b_val, W_out, b_out):
    del spatial_shapes
    refx = reference_points[..., 0]
    refy = reference_points[..., 1]
    Wox, Woy = W_off[:, 0::2], W_off[:, 1::2]
    box, boy = b_off[0::2][None, :], b_off[1::2][None, :]
    value, idx, wt = _prep(query, refx, refy, W_val[:, _LOHI],
                           b_val[_LOHI][None, :], Wox, box,
                           Woy, boy, W_attn, b_attn[None, :])
    attn = _make_sample()(value.reshape(BS * LQ * NH, 16),
                          idx.reshape(BS * LQ * 4, K128),
                          wt.reshape(BS * LQ, KT))
    out = _outproj(attn, query.reshape(BS * LQ, D), W_out, b_out[None, :])
    return out.reshape(BS, LQ, D)


# speedup vs baseline: 5.5289x; 1.0303x over previous
import functools

import jax
import jax.numpy as jnp
import numpy as np
from jax import lax
from jax.experimental import pallas as pl
from jax.experimental.pallas import tpu as pltpu
from jax.experimental.pallas import tpu_sc as plsc

BS, LQ, D = 4, 5440, 256
NH, NL, NPT = 8, 4, 4
HD = D // NH
SSH = np.array([[64, 64], [32, 32], [16, 16], [8, 8]], dtype=np.int64)
STARTS = np.concatenate([[0], np.cumsum(SSH[:, 0] * SSH[:, 1])])
K128 = NH * NL * NPT
KT = 4 * K128
BLK = 544
NBLK = LQ // BLK

_l_of_k = (np.arange(K128) // NPT) % NL
_W_of_k = SSH[_l_of_k, 1].astype(np.float32)
_H_of_k = SSH[_l_of_k, 0].astype(np.float32)
_START_of_k = STARTS[_l_of_k].astype(np.int32)
_H_head_of_k = (np.arange(K128) // (NL * NPT)).astype(np.int32)
_PX = np.zeros((NL, K128), np.float32)
_PX[_l_of_k, np.arange(K128)] = _W_of_k
_PY = np.zeros((NL, K128), np.float32)
_PY[_l_of_k, np.arange(K128)] = _H_of_k
_G = np.kron(np.eye(NH, dtype=np.float32), np.ones((NL * NPT, NL * NPT), np.float32))


def _prep_body(q_ref, rx_ref, ry_ref, wval_ref, bval_ref, wox_ref, box_ref,
               woy_ref, boy_ref, wattn_ref, battn_ref, g_ref, px_ref, py_ref,
               tabf_ref, tabi_ref,
               val_ref, idx_ref, wt_ref):
    b = pl.program_id(0)
    q = q_ref[0]
    val = jnp.dot(q, wval_ref[...], preferred_element_type=jnp.float32, precision=lax.Precision.HIGHEST) + bval_ref[...]
    u = lax.bitcast_convert_type(val, jnp.int32)
    rnd = lambda x: lax.shift_right_logical(
        x + 0x7FFF + (lax.shift_right_logical(x, 16) & 1), 16)
    val_ref[0] = rnd(u[:, :K128]) | (rnd(u[:, K128:]) << 16)
    offx = jnp.dot(q, wox_ref[...], preferred_element_type=jnp.float32, precision=lax.Precision.HIGHEST) + box_ref[...]
    offy = jnp.dot(q, woy_ref[...], preferred_element_type=jnp.float32, precision=lax.Precision.HIGHEST) + boy_ref[...]
    a = jnp.dot(q, wattn_ref[...], preferred_element_type=jnp.float32, precision=lax.Precision.HIGHEST) + battn_ref[...]
    a = a - jnp.max(a, axis=-1, keepdims=True)
    e = jnp.exp(a)
    aw = e / jnp.dot(e, g_ref[...], preferred_element_type=jnp.float32, precision=lax.Precision.HIGHEST)

    px = jnp.dot(rx_ref[0], px_ref[...], preferred_element_type=jnp.float32, precision=lax.Precision.HIGHEST) + offx - 0.5
    py = jnp.dot(ry_ref[0], py_ref[...], preferred_element_type=jnp.float32, precision=lax.Precision.HIGHEST) + offy - 0.5
    x0 = jnp.floor(px)
    y0 = jnp.floor(py)
    fx = px - x0
    fy = py - y0

    wc = tabf_ref[0:1, :]
    hc = tabf_ref[1:2, :]
    wci = tabi_ref[0:1, :]
    sc = tabi_ref[1:2, :]
    hk = tabi_ref[2:3, :]
    for c in range(4):
        dx, dy = float(c & 1), float(c >> 1)
        ix = x0 + dx
        iy = y0 + dy
        valid = ((ix >= 0.0) & (ix <= wc - 1.0) & (iy >= 0.0) & (iy <= hc - 1.0))
        ixc = jnp.clip(ix, 0.0, wc - 1.0).astype(jnp.int32)
        iyc = jnp.clip(iy, 0.0, hc - 1.0).astype(jnp.int32)
        pos = sc + iyc * wci + ixc
        row = (b * LQ + pos) * NH + hk
        wx = fx if dx else 1.0 - fx
        wy = fy if dy else 1.0 - fy
        idx_ref[0, :, c * K128:(c + 1) * K128] = row
        wt_ref[0, :, c * K128:(c + 1) * K128] = aw * wx * wy * valid.astype(jnp.float32)


def _prep(query, refx, refy, W_val, b_val, Wox, box, Woy, boy, W_attn, b_attn):
    full = lambda shp: pl.BlockSpec(shp, lambda b, i: tuple(0 for _ in shp))
    return pl.pallas_call(
        _prep_body,
        grid=(BS, NBLK),
        in_specs=[
            pl.BlockSpec((1, BLK, D), lambda b, i: (b, i, 0)),
            pl.BlockSpec((1, BLK, NL), lambda b, i: (b, i, 0)),
            pl.BlockSpec((1, BLK, NL), lambda b, i: (b, i, 0)),
            full((D, D)), full((1, D)),
            full((D, K128)), full((1, K128)),
            full((D, K128)), full((1, K128)),
            full((D, K128)), full((1, K128)),
            full((K128, K128)), full((NL, K128)), full((NL, K128)),
            full((2, K128)), full((3, K128)),
        ],
        out_specs=[
            pl.BlockSpec((1, BLK, K128), lambda b, i: (b, i, 0)),
            pl.BlockSpec((1, BLK, KT), lambda b, i: (b, i, 0)),
            pl.BlockSpec((1, BLK, KT), lambda b, i: (b, i, 0)),
        ],
        out_shape=[
            jax.ShapeDtypeStruct((BS, LQ, K128), jnp.int32),
            jax.ShapeDtypeStruct((BS, LQ, KT), jnp.int32),
            jax.ShapeDtypeStruct((BS, LQ, KT), jnp.float32),
        ],
    )(query, refx, refy, W_val, b_val, Wox, box, Woy, boy, W_attn, b_attn,
      jnp.asarray(_G), jnp.asarray(_PX), jnp.asarray(_PY),
      jnp.asarray(np.stack([_W_of_k, _H_of_k])),
      jnp.asarray(np.stack([_W_of_k.astype(np.int32), _START_of_k, _H_head_of_k])))


NW = 32
QPW = (BS * LQ) // NW
CH = 4
NIT = QPW // CH


def _sample_body(value_hbm, idx_hbm, wt_hbm, out_hbm, idx_v, wt_v, rows_v, out_v,
                 gsem0, gsem1, psem):
    wid = lax.axis_index("s") * 2 + lax.axis_index("c")
    qbase0 = wid * QPW
    gsems = (gsem0, gsem1)

    def idxwt_descs(it, slot):
        qb = qbase0 + it * CH
        return (pltpu.make_async_copy(idx_hbm.at[pl.ds(qb * 4, CH * 4)],
                                      idx_v.at[slot], psem),
                pltpu.make_async_copy(wt_hbm.at[pl.ds(qb, CH)],
                                      wt_v.at[slot], psem))

    def gather_descs(slot):
        return [pltpu.make_async_copy(value_hbm.at[idx_v.at[slot, j]],
                                      rows_v.at[slot, pl.ds(j * 128, 128)],
                                      gsems[slot])
                for j in range(CH * 4)]

    def compute(it, slot):
        for qq in range(CH):
            def h_body(h, c2):
                acc = [jnp.zeros((16,), jnp.float32) for _ in range(4)]
                for c in range(4):
                    wv = wt_v[slot, qq, pl.ds(c * K128 + h * 16, 16)]
                    for t in range(16):
                        r = qq * KT + c * K128 + h * 16 + t
                        v = rows_v[slot, r, :]
                        lo = lax.bitcast_convert_type(v << 16, jnp.float32)
                        hi = lax.bitcast_convert_type(v, jnp.float32)
                        p = t & 1
                        acc[p] = acc[p] + wv[t] * lo
                        acc[2 + p] = acc[2 + p] + wv[t] * hi
                out_v[qq, pl.ds(h * HD, 16)] = acc[0] + acc[1]
                out_v[qq, pl.ds(h * HD + 16, 16)] = acc[2] + acc[3]
                return c2
            lax.fori_loop(0, NH, h_body, 0)
        pltpu.sync_copy(out_v, out_hbm.at[pl.ds(qbase0 + it * CH, CH)])

    pltpu.sync_copy(idx_hbm.at[pl.ds(qbase0 * 4, CH * 4)], idx_v.at[0])
    pltpu.sync_copy(wt_hbm.at[pl.ds(qbase0, CH)], wt_v.at[0])
    for d in gather_descs(0):
        d.start()
    for d in idxwt_descs(1, 1):
        d.start()

    def body2(it2, carry):
        for slot in (0, 1):
            it = it2 * 2 + slot
            nxt = 1 - slot

            @pl.when(it + 1 < NIT)
            def _():
                for d in idxwt_descs(it + 1, nxt):
                    d.wait()
                for d in gather_descs(nxt):
                    d.start()

            for d in gather_descs(slot):
                d.wait()

            compute(it, slot)

            @pl.when(it + 2 < NIT)
            def _():
                for d in idxwt_descs(it + 2, slot):
                    d.start()
        return carry

    lax.fori_loop(0, NIT // 2, body2, 0)


@functools.cache
def _make_sample():
    return pl.kernel(
        _sample_body,
        mesh=plsc.VectorSubcoreMesh(core_axis_name="c", subcore_axis_name="s"),
        compiler_params=pltpu.CompilerParams(use_tc_tiling_on_sc=False),
        out_type=jax.ShapeDtypeStruct((BS * LQ, D), jnp.float32),
        scratch_types=[
            pltpu.VMEM((2, CH * 4, 128), jnp.int32),
            pltpu.VMEM((2, CH, KT), jnp.float32),
            pltpu.VMEM((2, CH * KT, 16), jnp.int32),
            pltpu.VMEM((CH, D), jnp.float32),
            pltpu.SemaphoreType.DMA,
            pltpu.SemaphoreType.DMA,
            pltpu.SemaphoreType.DMA,
        ],
    )


def _outproj_body(a_ref, q_ref, w_ref, b_ref, o_ref):
    o_ref[...] = (q_ref[...] + b_ref[...]
                  + jnp.dot(a_ref[...], w_ref[...], preferred_element_type=jnp.float32, precision=lax.Precision.HIGHEST))


def _outproj(attn, query2, W_out, b_out):
    return pl.pallas_call(
        _outproj_body,
        grid=(BS * NBLK,),
        in_specs=[
            pl.BlockSpec((BLK, D), lambda i: (i, 0)),
            pl.BlockSpec((BLK, D), lambda i: (i, 0)),
            pl.BlockSpec((D, D), lambda i: (0, 0)),
            pl.BlockSpec((1, D), lambda i: (0, 0)),
        ],
        out_specs=pl.BlockSpec((BLK, D), lambda i: (i, 0)),
        out_shape=jax.ShapeDtypeStruct((BS * LQ, D), jnp.float32),
    )(attn, query2, W_out, b_out)


_LOHI = np.concatenate([np.arange(NH * HD).reshape(NH, 2, 16)[:, 0].ravel(),
                        np.arange(NH * HD).reshape(NH, 2, 16)[:, 1].ravel()])


def kernel(query, reference_points, spatial_shapes, W_off, b_off, W_attn, b_attn,
           W_val, b_val, W_out, b_out):
    del spatial_shapes
    refx = reference_points[..., 0]
    refy = reference_points[..., 1]
    Wox, Woy = W_off[:, 0::2], W_off[:, 1::2]
    box, boy = b_off[0::2][None, :], b_off[1::2][None, :]
    value, idx, wt = _prep(query, refx, refy, W_val[:, _LOHI],
                           b_val[_LOHI][None, :], Wox, box,
                           Woy, boy, W_attn, b_attn[None, :])
    attn = _make_sample()(value.reshape(BS * LQ * NH, 16),
                          idx.reshape(BS * LQ * 4, K128),
                          wt.reshape(BS * LQ, KT))
    out = _outproj(attn, query.reshape(BS * LQ, D), W_out, b_out[None, :])
    return out.reshape(BS, LQ, D)


# speedup vs baseline: 5.6346x; 1.0191x over previous
import functools

import jax
import jax.numpy as jnp
import numpy as np
from jax import lax
from jax.experimental import pallas as pl
from jax.experimental.pallas import tpu as pltpu
from jax.experimental.pallas import tpu_sc as plsc

BS, LQ, D = 4, 5440, 256
NH, NL, NPT = 8, 4, 4
HD = D // NH
SSH = np.array([[64, 64], [32, 32], [16, 16], [8, 8]], dtype=np.int64)
STARTS = np.concatenate([[0], np.cumsum(SSH[:, 0] * SSH[:, 1])])
K128 = NH * NL * NPT
KT = 4 * K128
BLK = 544
NBLK = LQ // BLK

_l_of_k = (np.arange(K128) // NPT) % NL
_W_of_k = SSH[_l_of_k, 1].astype(np.float32)
_H_of_k = SSH[_l_of_k, 0].astype(np.float32)
_START_of_k = STARTS[_l_of_k].astype(np.int32)
_H_head_of_k = (np.arange(K128) // (NL * NPT)).astype(np.int32)
_PX = np.zeros((NL, K128), np.float32)
_PX[_l_of_k, np.arange(K128)] = _W_of_k
_PY = np.zeros((NL, K128), np.float32)
_PY[_l_of_k, np.arange(K128)] = _H_of_k
_G = np.kron(np.eye(NH, dtype=np.float32), np.ones((NL * NPT, NL * NPT), np.float32))


def _prep_body(q_ref, rx_ref, ry_ref, wval_ref, bval_ref, wox_ref, box_ref,
               woy_ref, boy_ref, wattn_ref, battn_ref, g_ref, px_ref, py_ref,
               tabf_ref, tabi_ref,
               val_ref, idx_ref, wt_ref):
    b = pl.program_id(0)
    q = q_ref[0]
    val = jnp.dot(q, wval_ref[...], preferred_element_type=jnp.float32, precision=lax.Precision.HIGHEST) + bval_ref[...]
    u = lax.bitcast_convert_type(val, jnp.int32)
    rnd = lambda x: lax.shift_right_logical(
        x + 0x7FFF + (lax.shift_right_logical(x, 16) & 1), 16)
    val_ref[0] = rnd(u[:, :K128]) | (rnd(u[:, K128:]) << 16)
    offx = jnp.dot(q, wox_ref[...], preferred_element_type=jnp.float32, precision=lax.Precision.HIGHEST) + box_ref[...]
    offy = jnp.dot(q, woy_ref[...], preferred_element_type=jnp.float32, precision=lax.Precision.HIGHEST) + boy_ref[...]
    a = jnp.dot(q, wattn_ref[...], preferred_element_type=jnp.float32, precision=lax.Precision.HIGHEST) + battn_ref[...]
    a = a - jnp.max(a, axis=-1, keepdims=True)
    e = jnp.exp(a)
    aw = e / jnp.dot(e, g_ref[...], preferred_element_type=jnp.float32, precision=lax.Precision.HIGHEST)

    px = jnp.dot(rx_ref[0], px_ref[...], preferred_element_type=jnp.float32, precision=lax.Precision.HIGHEST) + offx - 0.5
    py = jnp.dot(ry_ref[0], py_ref[...], preferred_element_type=jnp.float32, precision=lax.Precision.HIGHEST) + offy - 0.5
    x0 = jnp.floor(px)
    y0 = jnp.floor(py)
    fx = px - x0
    fy = py - y0

    wc = tabf_ref[0:1, :]
    hc = tabf_ref[1:2, :]
    wci = tabi_ref[0:1, :]
    sc = tabi_ref[1:2, :]
    hk = tabi_ref[2:3, :]
    for c in range(4):
        dx, dy = float(c & 1), float(c >> 1)
        ix = x0 + dx
        iy = y0 + dy
        valid = ((ix >= 0.0) & (ix <= wc - 1.0) & (iy >= 0.0) & (iy <= hc - 1.0))
        ixc = jnp.clip(ix, 0.0, wc - 1.0).astype(jnp.int32)
        iyc = jnp.clip(iy, 0.0, hc - 1.0).astype(jnp.int32)
        pos = sc + iyc * wci + ixc
        row = (b * LQ + pos) * NH + hk
        wx = fx if dx else 1.0 - fx
        wy = fy if dy else 1.0 - fy
        idx_ref[0, :, c * K128:(c + 1) * K128] = row
        wt_ref[0, :, c * K128:(c + 1) * K128] = aw * wx * wy * valid.astype(jnp.float32)


def _prep(query, refx, refy, W_val, b_val, Wox, box, Woy, boy, W_attn, b_attn):
    full = lambda shp: pl.BlockSpec(shp, lambda b, i: tuple(0 for _ in shp))
    return pl.pallas_call(
        _prep_body,
        grid=(BS, NBLK),
        in_specs=[
            pl.BlockSpec((1, BLK, D), lambda b, i: (b, i, 0)),
            pl.BlockSpec((1, BLK, NL), lambda b, i: (b, i, 0)),
            pl.BlockSpec((1, BLK, NL), lambda b, i: (b, i, 0)),
            full((D, D)), full((1, D)),
            full((D, K128)), full((1, K128)),
            full((D, K128)), full((1, K128)),
            full((D, K128)), full((1, K128)),
            full((K128, K128)), full((NL, K128)), full((NL, K128)),
            full((2, K128)), full((3, K128)),
        ],
        out_specs=[
            pl.BlockSpec((1, BLK, K128), lambda b, i: (b, i, 0)),
            pl.BlockSpec((1, BLK, KT), lambda b, i: (b, i, 0)),
            pl.BlockSpec((1, BLK, KT), lambda b, i: (b, i, 0)),
        ],
        out_shape=[
            jax.ShapeDtypeStruct((BS, LQ, K128), jnp.int32),
            jax.ShapeDtypeStruct((BS, LQ, KT), jnp.int32),
            jax.ShapeDtypeStruct((BS, LQ, KT), jnp.float32),
        ],
    )(query, refx, refy, W_val, b_val, Wox, box, Woy, boy, W_attn, b_attn,
      jnp.asarray(_G), jnp.asarray(_PX), jnp.asarray(_PY),
      jnp.asarray(np.stack([_W_of_k, _H_of_k])),
      jnp.asarray(np.stack([_W_of_k.astype(np.int32), _START_of_k, _H_head_of_k])))


NW = 32
QPW = (BS * LQ) // NW
CH = 5
NIT = QPW // CH


def _sample_body(value_hbm, idx_hbm, wt_hbm, out_hbm, idx_v, wt_v, rows_v, out_v,
                 gsem0, gsem1, psem):
    wid = lax.axis_index("s") * 2 + lax.axis_index("c")
    qbase0 = wid * QPW
    gsems = (gsem0, gsem1)

    def idxwt_descs(it, slot):
        qb = qbase0 + it * CH
        return (pltpu.make_async_copy(idx_hbm.at[pl.ds(qb * 4, CH * 4)],
                                      idx_v.at[slot], psem),
                pltpu.make_async_copy(wt_hbm.at[pl.ds(qb, CH)],
                                      wt_v.at[slot], psem))

    def gather_descs(slot):
        return [pltpu.make_async_copy(value_hbm.at[idx_v.at[slot, j]],
                                      rows_v.at[slot, pl.ds(j * 128, 128)],
                                      gsems[slot])
                for j in range(CH * 4)]

    def compute(it, slot):
        for qq in range(CH):
            def h_body(h, c2):
                acc = [jnp.zeros((16,), jnp.float32) for _ in range(4)]
                for c in range(4):
                    wv = wt_v[slot, qq, pl.ds(c * K128 + h * 16, 16)]
                    for t in range(16):
                        r = qq * KT + c * K128 + h * 16 + t
                        v = rows_v[slot, r, :]
                        lo = lax.bitcast_convert_type(v << 16, jnp.float32)
                        hi = lax.bitcast_convert_type(v, jnp.float32)
                        p = t & 1
                        acc[p] = acc[p] + wv[t] * lo
                        acc[2 + p] = acc[2 + p] + wv[t] * hi
                out_v[qq, pl.ds(h * HD, 16)] = acc[0] + acc[1]
                out_v[qq, pl.ds(h * HD + 16, 16)] = acc[2] + acc[3]
                return c2
            lax.fori_loop(0, NH, h_body, 0)
        pltpu.sync_copy(out_v, out_hbm.at[pl.ds(qbase0 + it * CH, CH)])

    pltpu.sync_copy(idx_hbm.at[pl.ds(qbase0 * 4, CH * 4)], idx_v.at[0])
    pltpu.sync_copy(wt_hbm.at[pl.ds(qbase0, CH)], wt_v.at[0])
    for d in gather_descs(0):
        d.start()
    for d in idxwt_descs(1, 1):
        d.start()

    def body2(it2, carry):
        for slot in (0, 1):
            it = it2 * 2 + slot
            nxt = 1 - slot

            @pl.when(it + 1 < NIT)
            def _():
                for d in idxwt_descs(it + 1, nxt):
                    d.wait()
                for d in gather_descs(nxt):
                    d.start()

            for d in gather_descs(slot):
                d.wait()

            compute(it, slot)

            @pl.when(it + 2 < NIT)
            def _():
                for d in idxwt_descs(it + 2, slot):
                    d.start()
        return carry

    lax.fori_loop(0, NIT // 2, body2, 0)


@functools.cache
def _make_sample():
    return pl.kernel(
        _sample_body,
        mesh=plsc.VectorSubcoreMesh(core_axis_name="c", subcore_axis_name="s"),
        compiler_params=pltpu.CompilerParams(use_tc_tiling_on_sc=False),
        out_type=jax.ShapeDtypeStruct((BS * LQ, D), jnp.float32),
        scratch_types=[
            pltpu.VMEM((2, CH * 4, 128), jnp.int32),
            pltpu.VMEM((2, CH, KT), jnp.float32),
            pltpu.VMEM((2, CH * KT, 16), jnp.int32),
            pltpu.VMEM((CH, D), jnp.float32),
            pltpu.SemaphoreType.DMA,
            pltpu.SemaphoreType.DMA,
            pltpu.SemaphoreType.DMA,
        ],
    )


def _outproj_body(a_ref, q_ref, w_ref, b_ref, o_ref):
    o_ref[...] = (q_ref[...] + b_ref[...]
                  + jnp.dot(a_ref[...], w_ref[...], preferred_element_type=jnp.float32, precision=lax.Precision.HIGHEST))


def _outproj(attn, query2, W_out, b_out):
    return pl.pallas_call(
        _outproj_body,
        grid=(BS * NBLK,),
        in_specs=[
            pl.BlockSpec((BLK, D), lambda i: (i, 0)),
            pl.BlockSpec((BLK, D), lambda i: (i, 0)),
            pl.BlockSpec((D, D), lambda i: (0, 0)),
            pl.BlockSpec((1, D), lambda i: (0, 0)),
        ],
        out_specs=pl.BlockSpec((BLK, D), lambda i: (i, 0)),
        out_shape=jax.ShapeDtypeStruct((BS * LQ, D), jnp.float32),
    )(attn, query2, W_out, b_out)


_LOHI = np.concatenate([np.arange(NH * HD).reshape(NH, 2, 16)[:, 0].ravel(),
                        np.arange(NH * HD).reshape(NH, 2, 16)[:, 1].ravel()])


def kernel(query, reference_points, spatial_shapes, W_off, b_off, W_attn, b_attn,
           W_val, b_val, W_out, b_out):
    del spatial_shapes
    refx = reference_points[..., 0]
    refy = reference_points[..., 1]
    Wox, Woy = W_off[:, 0::2], W_off[:, 1::2]
    box, boy = b_off[0::2][None, :], b_off[1::2][None, :]
    value, idx, wt = _prep(query, refx, refy, W_val[:, _LOHI],
                           b_val[_LOHI][None, :], Wox, box,
                           Woy, boy, W_attn, b_attn[None, :])
    attn = _make_sample()(value.reshape(BS * LQ * NH, 16),
                          idx.reshape(BS * LQ * 4, K128),
                          wt.reshape(BS * LQ, KT))
    out = _outproj(attn, query.reshape(BS * LQ, D), W_out, b_out[None, :])
    return out.reshape(BS, LQ, D)
